# baseline (speedup 1.0000x reference)
import jax
import jax.numpy as jnp
from jax.experimental import pallas as pl
from jax.experimental.pallas import tpu as pltpu

N = 100000
F = 128
G = 512
TILE = 2000
GRID = N // TILE
WIN = 128


def _body(meta_ref, x_ref, i_ref, a_ref, o_ref, acc_ref, m_ref, d_ref):
    step = pl.program_id(0)

    @pl.when(step == 0)
    def _init():
        m_ref[0, 0] = -jnp.inf
        d_ref[0, 0] = 0.0
        acc_ref[...] = jnp.zeros_like(acc_ref)

    x = x_ref[...]
    logits = jnp.dot(x, a_ref[...], preferred_element_type=jnp.float32)
    m_old = m_ref[0, 0]
    m_new = jnp.maximum(m_old, jnp.max(logits))
    m_ref[0, 0] = m_new
    scale = jnp.exp(m_old - m_new)
    w = jnp.exp(logits - m_new)
    d_ref[0, 0] = d_ref[0, 0] * scale + jnp.sum(w)

    @pl.when(m_new > m_old)
    def _rescale():
        acc_ref[...] = acc_ref[...] * scale

    iv = i_ref[...]
    t = (w * x).astype(jnp.bfloat16)
    goff = meta_ref[step, 0]
    ok = meta_ref[step, 1] != 0

    @pl.when(ok)
    def _windowed():
        cols = jax.lax.broadcasted_iota(jnp.int16, (1, WIN), 1)
        local = iv - goff.astype(jnp.int16)
        p = jnp.where(local == cols, jnp.bfloat16(1), jnp.bfloat16(0))
        contrib = jax.lax.dot_general(
            p, t, (((0,), (0,)), ((), ())), preferred_element_type=jnp.float32)
        acc_ref[pl.ds(goff, WIN), :] = acc_ref[pl.ds(goff, WIN), :] + contrib

    @pl.when(jnp.logical_not(ok))
    def _full():
        cols = jax.lax.broadcasted_iota(jnp.int16, (1, G), 1)
        p = jnp.where(iv == cols, jnp.bfloat16(1), jnp.bfloat16(0))
        contrib = jax.lax.dot_general(
            p, t, (((0,), (0,)), ((), ())), preferred_element_type=jnp.float32)
        acc_ref[...] = acc_ref[...] + contrib

    @pl.when(step == GRID - 1)
    def _finish():
        o_ref[...] = acc_ref[...] / d_ref[0, 0]


def kernel(X, I, attn_kernel):
    Ii = I.astype(jnp.int32)
    starts = Ii[::TILE]
    ends = Ii[TILE - 1::TILE]
    goff = jnp.minimum((starts // 8) * 8, G - WIN)
    ok = (ends - goff) < WIN
    meta = jnp.stack([goff, ok.astype(jnp.int32)], axis=1)
    I2 = Ii.astype(jnp.int16).reshape(N, 1)
    return pl.pallas_call(
        _body,
        grid=(GRID,),
        in_specs=[
            pl.BlockSpec(memory_space=pltpu.SMEM),
            pl.BlockSpec((TILE, F), lambda i: (i, 0)),
            pl.BlockSpec((TILE, 1), lambda i: (i, 0)),
            pl.BlockSpec((F, 1), lambda i: (0, 0)),
        ],
        out_specs=pl.BlockSpec((G, F), lambda i: (0, 0)),
        out_shape=jax.ShapeDtypeStruct((G, F), jnp.float32),
        scratch_shapes=[
            pltpu.VMEM((G, F), jnp.float32),
            pltpu.SMEM((1, 1), jnp.float32),
            pltpu.SMEM((1, 1), jnp.float32),
        ],
        compiler_params=pltpu.CompilerParams(
            dimension_semantics=("arbitrary",),
        ),
    )(meta, X, I2, attn_kernel)


# speedup vs baseline: 1.1061x; 1.1061x over previous
import jax
import jax.numpy as jnp
from jax.experimental import pallas as pl
from jax.experimental.pallas import tpu as pltpu

N = 100000
F = 128
G = 512
TILE = 2000
GRID = N // TILE


def _body(x_ref, i_ref, a_ref, o_ref, acc_ref, m_ref, d_ref):
    step = pl.program_id(0)

    @pl.when(step == 0)
    def _init():
        m_ref[0, 0] = -jnp.inf
        d_ref[0, 0] = 0.0
        acc_ref[...] = jnp.zeros_like(acc_ref)

    x = x_ref[...]
    logits = jnp.dot(x, a_ref[...], preferred_element_type=jnp.float32)
    m_old = m_ref[0, 0]
    m_new = jnp.maximum(m_old, jnp.max(logits))
    m_ref[0, 0] = m_new
    scale = jnp.exp(m_old - m_new)
    w = jnp.exp(logits - m_new)
    d_ref[0, 0] = d_ref[0, 0] * scale + jnp.sum(w)

    cols = jax.lax.broadcasted_iota(jnp.int16, (1, G), 1)
    p = jnp.where(i_ref[...] == cols,
                  jnp.bfloat16(1), jnp.bfloat16(0))
    t = (w * x).astype(jnp.bfloat16)
    contrib = jax.lax.dot_general(
        p, t, (((0,), (0,)), ((), ())), preferred_element_type=jnp.float32)

    @pl.when(m_new > m_old)
    def _rescale_add():
        acc_ref[...] = acc_ref[...] * scale + contrib

    @pl.when(jnp.logical_not(m_new > m_old))
    def _plain_add():
        acc_ref[...] = acc_ref[...] + contrib

    @pl.when(step == GRID - 1)
    def _finish():
        o_ref[...] = acc_ref[...] / d_ref[0, 0]


def kernel(X, I, attn_kernel):
    I2 = I.astype(jnp.int16).reshape(N, 1)
    return pl.pallas_call(
        _body,
        grid=(GRID,),
        in_specs=[
            pl.BlockSpec((TILE, F), lambda i: (i, 0)),
            pl.BlockSpec((TILE, 1), lambda i: (i, 0)),
            pl.BlockSpec((F, 1), lambda i: (0, 0)),
        ],
        out_specs=pl.BlockSpec((G, F), lambda i: (0, 0)),
        out_shape=jax.ShapeDtypeStruct((G, F), jnp.float32),
        scratch_shapes=[
            pltpu.VMEM((G, F), jnp.float32),
            pltpu.SMEM((1, 1), jnp.float32),
            pltpu.SMEM((1, 1), jnp.float32),
        ],
        compiler_params=pltpu.CompilerParams(
            dimension_semantics=("arbitrary",),
        ),
    )(X, I2, attn_kernel)


# speedup vs baseline: 1.7883x; 1.6167x over previous
import jax
import jax.numpy as jnp
from jax.experimental import pallas as pl
from jax.experimental.pallas import tpu as pltpu

N = 100000
F = 128
G = 512
TILE = 2000
GRID = N // TILE
WIN = 128


def _body(meta_ref, x_ref, i_ref, a_ref, o_ref, acc_ref, m_ref, d_ref):
    step = pl.program_id(0)

    @pl.when(step == 0)
    def _init():
        m_ref[0, 0] = -jnp.inf
        d_ref[0, 0] = 0.0
        acc_ref[...] = jnp.zeros_like(acc_ref)

    x = x_ref[...]
    logits = jnp.dot(x, a_ref[...], preferred_element_type=jnp.float32)
    m_old = m_ref[0, 0]
    m_new = jnp.maximum(m_old, jnp.max(logits))
    m_ref[0, 0] = m_new
    scale = jnp.exp(m_old - m_new)
    w = jnp.exp(logits - m_new)
    d_ref[0, 0] = d_ref[0, 0] * scale + jnp.sum(w)

    @pl.when(m_new > m_old)
    def _rescale():
        acc_ref[...] = acc_ref[...] * scale

    iv = i_ref[0]
    t = (w * x).astype(jnp.bfloat16)
    goff = meta_ref[step, 0]
    ok = meta_ref[step, 1] != 0

    @pl.when(ok)
    def _windowed():
        rows = jax.lax.broadcasted_iota(jnp.int16, (WIN, 1), 0)
        local = (iv - goff).astype(jnp.int16)
        pt = jnp.where(rows == local, jnp.bfloat16(1), jnp.bfloat16(0))
        contrib = jax.lax.dot_general(
            pt, t, (((1,), (0,)), ((), ())), preferred_element_type=jnp.float32)
        acc_ref[pl.ds(goff, WIN), :] = acc_ref[pl.ds(goff, WIN), :] + contrib

    @pl.when(jnp.logical_not(ok))
    def _full():
        rows = jax.lax.broadcasted_iota(jnp.int16, (G, 1), 0)
        pt = jnp.where(rows == iv.astype(jnp.int16),
                       jnp.bfloat16(1), jnp.bfloat16(0))
        contrib = jax.lax.dot_general(
            pt, t, (((1,), (0,)), ((), ())), preferred_element_type=jnp.float32)
        acc_ref[...] = acc_ref[...] + contrib

    @pl.when(step == GRID - 1)
    def _finish():
        o_ref[...] = acc_ref[...] / d_ref[0, 0]


def kernel(X, I, attn_kernel):
    Ii = I.astype(jnp.int32)
    starts = Ii[::TILE]
    ends = Ii[TILE - 1::TILE]
    goff = jnp.minimum((starts // 8) * 8, G - WIN)
    ok = (ends - goff) < WIN
    meta = jnp.stack([goff, ok.astype(jnp.int32)], axis=1)
    I3 = Ii.reshape(GRID, 1, TILE)
    return pl.pallas_call(
        _body,
        grid=(GRID,),
        in_specs=[
            pl.BlockSpec(memory_space=pltpu.SMEM),
            pl.BlockSpec((TILE, F), lambda i: (i, 0)),
            pl.BlockSpec((1, 1, TILE), lambda i: (i, 0, 0)),
            pl.BlockSpec((F, 1), lambda i: (0, 0)),
        ],
        out_specs=pl.BlockSpec((G, F), lambda i: (0, 0)),
        out_shape=jax.ShapeDtypeStruct((G, F), jnp.float32),
        scratch_shapes=[
            pltpu.VMEM((G, F), jnp.float32),
            pltpu.SMEM((1, 1), jnp.float32),
            pltpu.SMEM((1, 1), jnp.float32),
        ],
        compiler_params=pltpu.CompilerParams(
            dimension_semantics=("arbitrary",),
        ),
    )(meta, X, I3, attn_kernel)


# speedup vs baseline: 2.4300x; 1.3588x over previous
import jax
import jax.numpy as jnp
from jax.experimental import pallas as pl
from jax.experimental.pallas import tpu as pltpu

N = 100000
F = 128
G = 512
TILE = 2000
GRID = N // TILE
WIN = 128


def _body(meta_ref, x_ref, i_ref, a_ref, o_ref, acc_ref, m_ref, d_ref):
    step = pl.program_id(0)

    @pl.when(step == 0)
    def _init():
        m_ref[0, 0] = -jnp.inf
        d_ref[0, 0] = 0.0
        acc_ref[...] = jnp.zeros_like(acc_ref)

    x = x_ref[...]
    logits = jax.lax.dot_general(
        a_ref[...], x, (((0,), (1,)), ((), ())),
        preferred_element_type=jnp.float32)
    m_old = m_ref[0, 0]
    m_new = jnp.maximum(m_old, jnp.max(logits))
    m_ref[0, 0] = m_new
    scale = jnp.exp(m_old - m_new)
    w = jnp.exp(logits - m_new)
    d_ref[0, 0] = d_ref[0, 0] * scale + jnp.sum(w)

    @pl.when(m_new > m_old)
    def _rescale():
        acc_ref[...] = acc_ref[...] * scale

    iv = i_ref[0]
    wb = w.astype(jnp.bfloat16)
    t = x.astype(jnp.bfloat16)
    goff = meta_ref[step, 0]
    ok = meta_ref[step, 1] != 0

    @pl.when(ok)
    def _windowed():
        rows = jax.lax.broadcasted_iota(jnp.int16, (WIN, 1), 0)
        local = (iv - goff).astype(jnp.int16)
        pt = jnp.where(rows == local, wb, jnp.bfloat16(0))
        contrib = jax.lax.dot_general(
            pt, t, (((1,), (0,)), ((), ())), preferred_element_type=jnp.float32)
        acc_ref[pl.ds(goff, WIN), :] = acc_ref[pl.ds(goff, WIN), :] + contrib

    @pl.when(jnp.logical_not(ok))
    def _full():
        rows = jax.lax.broadcasted_iota(jnp.int16, (G, 1), 0)
        pt = jnp.where(rows == iv.astype(jnp.int16),
                       wb, jnp.bfloat16(0))
        contrib = jax.lax.dot_general(
            pt, t, (((1,), (0,)), ((), ())), preferred_element_type=jnp.float32)
        acc_ref[...] = acc_ref[...] + contrib

    @pl.when(step == GRID - 1)
    def _finish():
        o_ref[...] = acc_ref[...] / d_ref[0, 0]


def kernel(X, I, attn_kernel):
    Ii = I.astype(jnp.int32)
    starts = Ii[::TILE]
    ends = Ii[TILE - 1::TILE]
    goff = jnp.minimum((starts // 8) * 8, G - WIN)
    ok = (ends - goff) < WIN
    meta = jnp.stack([goff, ok.astype(jnp.int32)], axis=1)
    I3 = Ii.reshape(GRID, 1, TILE)
    return pl.pallas_call(
        _body,
        grid=(GRID,),
        in_specs=[
            pl.BlockSpec(memory_space=pltpu.SMEM),
            pl.BlockSpec((TILE, F), lambda i: (i, 0)),
            pl.BlockSpec((1, 1, TILE), lambda i: (i, 0, 0)),
            pl.BlockSpec((F, 1), lambda i: (0, 0)),
        ],
        out_specs=pl.BlockSpec((G, F), lambda i: (0, 0)),
        out_shape=jax.ShapeDtypeStruct((G, F), jnp.float32),
        scratch_shapes=[
            pltpu.VMEM((G, F), jnp.float32),
            pltpu.SMEM((1, 1), jnp.float32),
            pltpu.SMEM((1, 1), jnp.float32),
        ],
        compiler_params=pltpu.CompilerParams(
            dimension_semantics=("arbitrary",),
        ),
    )(meta, X, I3, attn_kernel)


# speedup vs baseline: 3.5359x; 1.4551x over previous
import jax
import jax.numpy as jnp
from jax.experimental import pallas as pl
from jax.experimental.pallas import tpu as pltpu

N = 100000
F = 128
G = 512
TILE = 4000
GRID = N // TILE
WIN = 128


def _body(meta_ref, x_ref, i_ref, a_ref, o_ref, acc_ref, m_ref, d_ref):
    step = pl.program_id(0)

    @pl.when(step == 0)
    def _init():
        m_ref[0, 0] = -jnp.inf
        d_ref[0, 0] = 0.0
        acc_ref[...] = jnp.zeros_like(acc_ref)

    x = x_ref[...]
    logits = jax.lax.dot_general(
        a_ref[...], x, (((0,), (1,)), ((), ())),
        preferred_element_type=jnp.float32)
    m_old = m_ref[0, 0]
    m_new = jnp.maximum(m_old, jnp.max(logits))
    m_ref[0, 0] = m_new
    scale = jnp.exp(m_old - m_new)
    w = jnp.exp(logits - m_new)
    d_ref[0, 0] = d_ref[0, 0] * scale + jnp.sum(w)

    @pl.when(m_new > m_old)
    def _rescale():
        acc_ref[...] = acc_ref[...] * scale

    iv = i_ref[0]
    wb = w.astype(jnp.bfloat16)
    t = x.astype(jnp.bfloat16)
    goff = meta_ref[step, 0]
    ok = meta_ref[step, 1] != 0

    @pl.when(ok)
    def _windowed():
        rows = jax.lax.broadcasted_iota(jnp.int16, (WIN, 1), 0)
        local = (iv - goff).astype(jnp.int16)
        pt = jnp.where(rows == local, wb, jnp.bfloat16(0))
        contrib = jax.lax.dot_general(
            pt, t, (((1,), (0,)), ((), ())), preferred_element_type=jnp.float32)
        acc_ref[pl.ds(goff, WIN), :] = acc_ref[pl.ds(goff, WIN), :] + contrib

    @pl.when(jnp.logical_not(ok))
    def _full():
        rows = jax.lax.broadcasted_iota(jnp.int16, (G, 1), 0)
        pt = jnp.where(rows == iv.astype(jnp.int16),
                       wb, jnp.bfloat16(0))
        contrib = jax.lax.dot_general(
            pt, t, (((1,), (0,)), ((), ())), preferred_element_type=jnp.float32)
        acc_ref[...] = acc_ref[...] + contrib

    @pl.when(step == GRID - 1)
    def _finish():
        o_ref[...] = acc_ref[...] / d_ref[0, 0]


def kernel(X, I, attn_kernel):
    Ii = I.astype(jnp.int32)
    starts = Ii[::TILE]
    ends = Ii[TILE - 1::TILE]
    goff = jnp.minimum((starts // 8) * 8, G - WIN)
    ok = (ends - goff) < WIN
    meta = jnp.stack([goff, ok.astype(jnp.int32)], axis=1)
    I3 = Ii.reshape(GRID, 1, TILE)
    return pl.pallas_call(
        _body,
        grid=(GRID,),
        in_specs=[
            pl.BlockSpec(memory_space=pltpu.SMEM),
            pl.BlockSpec((TILE, F), lambda i: (i, 0)),
            pl.BlockSpec((1, 1, TILE), lambda i: (i, 0, 0)),
            pl.BlockSpec((F, 1), lambda i: (0, 0)),
        ],
        out_specs=pl.BlockSpec((G, F), lambda i: (0, 0)),
        out_shape=jax.ShapeDtypeStruct((G, F), jnp.float32),
        scratch_shapes=[
            pltpu.VMEM((G, F), jnp.float32),
            pltpu.SMEM((1, 1), jnp.float32),
            pltpu.SMEM((1, 1), jnp.float32),
        ],
        compiler_params=pltpu.CompilerParams(
            dimension_semantics=("arbitrary",),
        ),
    )(meta, X, I3, attn_kernel)


# speedup vs baseline: 4.7264x; 1.3367x over previous
import jax
import jax.numpy as jnp
from jax.experimental import pallas as pl
from jax.experimental.pallas import tpu as pltpu

N = 100000
F = 128
G = 512
TILE = 10000
GRID = N // TILE
WIN = 128


def _body(meta_ref, x_ref, i_ref, a_ref, o_ref, acc_ref, m_ref, d_ref):
    step = pl.program_id(0)

    @pl.when(step == 0)
    def _init():
        m_ref[0, 0] = -jnp.inf
        d_ref[0, 0] = 0.0
        acc_ref[...] = jnp.zeros_like(acc_ref)

    x = x_ref[...]
    logits = jax.lax.dot_general(
        a_ref[...], x, (((0,), (1,)), ((), ())),
        preferred_element_type=jnp.float32)
    m_old = m_ref[0, 0]
    m_new = jnp.maximum(m_old, jnp.max(logits))
    m_ref[0, 0] = m_new
    scale = jnp.exp(m_old - m_new)
    w = jnp.exp(logits - m_new)
    d_ref[0, 0] = d_ref[0, 0] * scale + jnp.sum(w)

    @pl.when(m_new > m_old)
    def _rescale():
        acc_ref[...] = acc_ref[...] * scale

    iv = i_ref[0]
    wb = w.astype(jnp.bfloat16)
    t = x.astype(jnp.bfloat16)
    goff = meta_ref[step, 0]
    ok = meta_ref[step, 1] != 0

    @pl.when(ok)
    def _windowed():
        rows = jax.lax.broadcasted_iota(jnp.int16, (WIN, 1), 0)
        local = (iv - goff).astype(jnp.int16)
        pt = jnp.where(rows == local, wb, jnp.bfloat16(0))
        contrib = jax.lax.dot_general(
            pt, t, (((1,), (0,)), ((), ())), preferred_element_type=jnp.float32)
        acc_ref[pl.ds(goff, WIN), :] = acc_ref[pl.ds(goff, WIN), :] + contrib

    @pl.when(jnp.logical_not(ok))
    def _full():
        rows = jax.lax.broadcasted_iota(jnp.int16, (G, 1), 0)
        pt = jnp.where(rows == iv.astype(jnp.int16),
                       wb, jnp.bfloat16(0))
        contrib = jax.lax.dot_general(
            pt, t, (((1,), (0,)), ((), ())), preferred_element_type=jnp.float32)
        acc_ref[...] = acc_ref[...] + contrib

    @pl.when(step == GRID - 1)
    def _finish():
        o_ref[...] = acc_ref[...] / d_ref[0, 0]


def kernel(X, I, attn_kernel):
    Ii = I.astype(jnp.int32)
    starts = Ii[::TILE]
    ends = Ii[TILE - 1::TILE]
    goff = jnp.minimum((starts // 8) * 8, G - WIN)
    ok = (ends - goff) < WIN
    meta = jnp.stack([goff, ok.astype(jnp.int32)], axis=1)
    I3 = Ii.reshape(GRID, 1, TILE)
    return pl.pallas_call(
        _body,
        grid=(GRID,),
        in_specs=[
            pl.BlockSpec(memory_space=pltpu.SMEM),
            pl.BlockSpec((TILE, F), lambda i: (i, 0)),
            pl.BlockSpec((1, 1, TILE), lambda i: (i, 0, 0)),
            pl.BlockSpec((F, 1), lambda i: (0, 0)),
        ],
        out_specs=pl.BlockSpec((G, F), lambda i: (0, 0)),
        out_shape=jax.ShapeDtypeStruct((G, F), jnp.float32),
        scratch_shapes=[
            pltpu.VMEM((G, F), jnp.float32),
            pltpu.SMEM((1, 1), jnp.float32),
            pltpu.SMEM((1, 1), jnp.float32),
        ],
        compiler_params=pltpu.CompilerParams(
            dimension_semantics=("arbitrary",),
        ),
    )(meta, X, I3, attn_kernel)


# speedup vs baseline: 5.1300x; 1.0854x over previous
import jax
import jax.numpy as jnp
from jax.experimental import pallas as pl
from jax.experimental.pallas import tpu as pltpu

N = 100000
F = 128
G = 512
TILE = 20000
GRID = N // TILE
WIN = 128


def _body(meta_ref, x_ref, i_ref, a_ref, o_ref, acc_ref, m_ref, d_ref):
    step = pl.program_id(0)

    @pl.when(step == 0)
    def _init():
        m_ref[0, 0] = -jnp.inf
        d_ref[0, 0] = 0.0
        acc_ref[...] = jnp.zeros_like(acc_ref)

    x = x_ref[...]
    logits = jax.lax.dot_general(
        a_ref[...], x, (((0,), (1,)), ((), ())),
        preferred_element_type=jnp.float32)
    m_old = m_ref[0, 0]
    m_new = jnp.maximum(m_old, jnp.max(logits))
    m_ref[0, 0] = m_new
    scale = jnp.exp(m_old - m_new)
    w = jnp.exp(logits - m_new)
    d_ref[0, 0] = d_ref[0, 0] * scale + jnp.sum(w)

    @pl.when(m_new > m_old)
    def _rescale():
        acc_ref[...] = acc_ref[...] * scale

    iv = i_ref[0]
    wb = w.astype(jnp.bfloat16)
    t = x.astype(jnp.bfloat16)
    goff = meta_ref[step, 0]
    ok = meta_ref[step, 1] != 0

    @pl.when(ok)
    def _windowed():
        rows = jax.lax.broadcasted_iota(jnp.int16, (WIN, 1), 0)
        local = (iv - goff).astype(jnp.int16)
        pt = jnp.where(rows == local, wb, jnp.bfloat16(0))
        contrib = jax.lax.dot_general(
            pt, t, (((1,), (0,)), ((), ())), preferred_element_type=jnp.float32)
        acc_ref[pl.ds(goff, WIN), :] = acc_ref[pl.ds(goff, WIN), :] + contrib

    @pl.when(jnp.logical_not(ok))
    def _full():
        rows = jax.lax.broadcasted_iota(jnp.int16, (G, 1), 0)
        pt = jnp.where(rows == iv.astype(jnp.int16),
                       wb, jnp.bfloat16(0))
        contrib = jax.lax.dot_general(
            pt, t, (((1,), (0,)), ((), ())), preferred_element_type=jnp.float32)
        acc_ref[...] = acc_ref[...] + contrib

    @pl.when(step == GRID - 1)
    def _finish():
        o_ref[...] = acc_ref[...] / d_ref[0, 0]


def kernel(X, I, attn_kernel):
    Ii = I.astype(jnp.int32)
    starts = Ii[::TILE]
    ends = Ii[TILE - 1::TILE]
    goff = jnp.minimum((starts // 8) * 8, G - WIN)
    ok = (ends - goff) < WIN
    meta = jnp.stack([goff, ok.astype(jnp.int32)], axis=1)
    I3 = Ii.reshape(GRID, 1, TILE)
    return pl.pallas_call(
        _body,
        grid=(GRID,),
        in_specs=[
            pl.BlockSpec(memory_space=pltpu.SMEM),
            pl.BlockSpec((TILE, F), lambda i: (i, 0)),
            pl.BlockSpec((1, 1, TILE), lambda i: (i, 0, 0)),
            pl.BlockSpec((F, 1), lambda i: (0, 0)),
        ],
        out_specs=pl.BlockSpec((G, F), lambda i: (0, 0)),
        out_shape=jax.ShapeDtypeStruct((G, F), jnp.float32),
        scratch_shapes=[
            pltpu.VMEM((G, F), jnp.float32),
            pltpu.SMEM((1, 1), jnp.float32),
            pltpu.SMEM((1, 1), jnp.float32),
        ],
        compiler_params=pltpu.CompilerParams(
            dimension_semantics=("arbitrary",),
        ),
    )(meta, X, I3, attn_kernel)
